# baseline (speedup 1.0000x reference)
import functools

import jax
import jax.numpy as jnp
from jax import lax
from jax.experimental import pallas as pl
from jax.experimental.pallas import tpu as pltpu
from jax.experimental.pallas import tpu_sc as plsc

_N, _D, _E, _G, _C = 10000, 128, 320000, 64, 10
_NACC = 10112
_CH = 128
_NW = 32
_CHW = 80
_CHP = 40
_NCHUNK = _NW * _CHW
_EPAD = _NCHUNK * _CH
_RPT = _NACC // 16


@functools.cache
def _sc_edge_aggregate():
    mesh = plsc.VectorSubcoreMesh(core_axis_name="c", subcore_axis_name="s")

    @functools.partial(
        pl.kernel,
        out_type=jax.ShapeDtypeStruct((2, _NACC, _D), jnp.float32),
        mesh=mesh,
        scratch_types=[
            pltpu.VMEM((_CHP, _CH), jnp.int32),
            pltpu.VMEM((_CHP, _CH), jnp.int32),
            pltpu.VMEM((_CH, _D), jnp.float32),
            pltpu.VMEM((_CH, _D), jnp.float32),
            pltpu.VMEM_SHARED((_NACC, _D), jnp.float32),
            pltpu.SemaphoreType.DMA,
            pltpu.SemaphoreType.DMA,
        ],
    )
    def body(zeros_hbm, src_hbm, dst_hbm, h_hbm, out_hbm,
             src_v, dst_v, rows0_v, rows1_v, acc_sh, sem0, sem1):
        c = lax.axis_index("c")
        s = lax.axis_index("s")
        wid = c * 16 + s
        pltpu.sync_copy(zeros_hbm.at[pl.ds(s * _RPT, _RPT)],
                        acc_sh.at[pl.ds(s * _RPT, _RPT)])
        plsc.subcore_barrier()

        bufs = (rows0_v, rows1_v)
        sems = (sem0, sem1)
        for p in range(2):
            base = wid * _CHW + p * _CHP
            pltpu.sync_copy(src_hbm.at[pl.ds(base, _CHP)], src_v)
            pltpu.sync_copy(dst_hbm.at[pl.ds(base, _CHP)], dst_v)
            pltpu.async_copy(h_hbm.at[src_v.at[0]], rows0_v, sem0)
            pltpu.async_copy(h_hbm.at[src_v.at[1]], rows1_v, sem1)

            def step(k, carry):
                for b in range(2):
                    j = 2 * k + b
                    buf, sem = bufs[b], sems[b]
                    pltpu.make_async_copy(
                        h_hbm.at[src_v.at[j]], buf, sem).wait()
                    pltpu.sync_copy(buf, acc_sh.at[dst_v.at[j]], add=True)

                    @pl.when(j + 2 < _CHP)
                    def _():
                        pltpu.async_copy(h_hbm.at[src_v.at[j + 2]], buf, sem)
                return carry

            lax.fori_loop(0, _CHP // 2, step, 0)
        plsc.subcore_barrier()
        pltpu.sync_copy(acc_sh.at[pl.ds(s * _RPT, _RPT)],
                        out_hbm.at[c, pl.ds(s * _RPT, _RPT)])

    return body


_NB = 10
_BLK = _N // _NB


def _mlp_body(x_ref, p_ref, w0_ref, b0_ref, w1_ref, b1_ref, o_ref):
    h = x_ref[...] + p_ref[0] + p_ref[1]
    t = jnp.dot(h, w0_ref[...], preferred_element_type=jnp.float32)
    t = jnp.maximum(t + b0_ref[...], 0.0)
    t = jnp.dot(t, w1_ref[...], preferred_element_type=jnp.float32)
    o_ref[...] = jnp.maximum(t + b1_ref[...], 0.0)


_mlp_call = pl.pallas_call(
    _mlp_body,
    grid=(_NB,),
    in_specs=[
        pl.BlockSpec((_BLK, _D), lambda i: (i, 0)),
        pl.BlockSpec((2, _BLK, _D), lambda i: (0, i, 0)),
        pl.BlockSpec((_D, _D), lambda i: (0, 0)),
        pl.BlockSpec((1, _D), lambda i: (0, 0)),
        pl.BlockSpec((_D, _D), lambda i: (0, 0)),
        pl.BlockSpec((1, _D), lambda i: (0, 0)),
    ],
    out_specs=pl.BlockSpec((_BLK, _D), lambda i: (i, 0)),
    out_shape=jax.ShapeDtypeStruct((_N, _D), jnp.float32),
)


def _pool_body(b_ref, h_ref, w_ref, bias_ref, o_ref, acc_ref):
    i = pl.program_id(0)

    @pl.when(i == 0)
    def _():
        acc_ref[...] = jnp.zeros_like(acc_ref)

    seg = b_ref[0, 0, :]
    oh = (seg[None, :] == lax.broadcasted_iota(jnp.int32, (_G, _BLK), 0))
    acc_ref[...] += jnp.dot(oh.astype(jnp.float32), h_ref[...],
                            preferred_element_type=jnp.float32)

    @pl.when(i == _NB - 1)
    def _():
        o_ref[...] = (jnp.dot(acc_ref[...], w_ref[...],
                              preferred_element_type=jnp.float32)
                      + bias_ref[...])


_pool_call = pl.pallas_call(
    _pool_body,
    grid=(_NB,),
    in_specs=[
        pl.BlockSpec((1, 1, _BLK), lambda i: (i, 0, 0)),
        pl.BlockSpec((_BLK, _D), lambda i: (i, 0)),
        pl.BlockSpec((_D, _C), lambda i: (0, 0)),
        pl.BlockSpec((1, _C), lambda i: (0, 0)),
    ],
    out_specs=pl.BlockSpec((_G, _C), lambda i: (0, 0)),
    out_shape=jax.ShapeDtypeStruct((_G, _C), jnp.float32),
    scratch_shapes=[pltpu.VMEM((_G, _D), jnp.float32)],
)


def kernel(x, edge_index, batch, Ws, bs, gammas, betas, lin_W, lin_b):
    inv = 1.0 / jnp.sqrt(1.0 + 1e-5)
    scale = gammas * inv
    Wf = Ws * scale[:, :, None, :]
    bf = bs * scale + betas

    src = edge_index[0]
    dst = edge_index[1]
    pad = _EPAD - _E
    src_p = jnp.concatenate([src, jnp.zeros((pad,), jnp.int32)]).reshape(
        _NCHUNK, _CH)
    dst_p = jnp.concatenate([dst, jnp.full((pad,), _N, jnp.int32)]).reshape(
        _NCHUNK, _CH)
    zeros = jnp.zeros((_NACC, _D), jnp.float32)

    h = x
    for i in range(4):
        parts = _sc_edge_aggregate()(zeros, src_p, dst_p, h)
        h = _mlp_call(h, parts, Wf[i, 0], bf[i, 0][None],
                      Wf[i, 1], bf[i, 1][None])
    batch3 = batch.reshape(_NB, 1, _BLK)
    return _pool_call(batch3, h, lin_W, lin_b[None])


# speedup vs baseline: 1.5643x; 1.5643x over previous
import functools

import jax
import jax.numpy as jnp
from jax import lax
from jax.experimental import pallas as pl
from jax.experimental.pallas import tpu as pltpu
from jax.experimental.pallas import tpu_sc as plsc

_N, _D, _E, _G, _C = 10000, 128, 320000, 64, 10
_NACC = 10112
_HALF = 5000
_TRASH = _NACC - 1
_CH = 32
_NW = 32
_CHW = 320
_PKW = 80
_NPK = 2560
_EPAD = _NPK * 128
_RPT = _NACC // 16


@functools.cache
def _sc_edge_aggregate():
    mesh = plsc.VectorSubcoreMesh(core_axis_name="c", subcore_axis_name="s")

    @functools.partial(
        pl.kernel,
        out_type=jax.ShapeDtypeStruct((2, _NACC, _D), jnp.float32),
        mesh=mesh,
        scratch_types=[
            pltpu.VMEM((8, 128), jnp.int32),
            pltpu.VMEM((_CH,), jnp.int32),
            pltpu.VMEM((_CH,), jnp.int32),
            pltpu.VMEM((_CH,), jnp.int32),
            pltpu.VMEM((_CH,), jnp.int32),
            pltpu.VMEM((_CH, _D), jnp.float32),
            pltpu.VMEM((_CH, _D), jnp.float32),
            pltpu.VMEM_SHARED((_NACC, _D), jnp.float32),
            pltpu.VMEM_SHARED((_HALF, _D), jnp.float32),
            pltpu.SemaphoreType.DMA,
            pltpu.SemaphoreType.DMA,
        ],
    )
    def body(zeros_hbm, pk_hbm, h_hbm, out_hbm,
             pk_v, srcs0, srcs1, dsts0, dsts1, buf0, buf1,
             acc_sh, slab_sh, sem0, sem1):
        c = lax.axis_index("c")
        s = lax.axis_index("s")
        wid = c * 16 + s
        srcs = (srcs0, srcs1)
        dsts = (dsts0, dsts1)
        bufs = (buf0, buf1)
        sems = (sem0, sem1)

        pltpu.sync_copy(zeros_hbm.at[pl.ds(s * _RPT, _RPT)],
                        acc_sh.at[pl.ds(s * _RPT, _RPT)])

        def unpack(j, b):
            r = (j // 4) % 8
            q = j % 4
            for v in range(2):
                w = pk_v[r, pl.ds(q * _CH + v * 16, 16)]
                srcs[b][pl.ds(v * 16, 16)] = w & 0x3FFF
                dsts[b][pl.ds(v * 16, 16)] = lax.shift_right_logical(w, 14)

        for p in range(2):
            plsc.subcore_barrier()
            pltpu.sync_copy(h_hbm.at[pl.ds(p * _HALF + s * 312, 312)],
                            slab_sh.at[pl.ds(s * 312, 312)])

            @pl.when(s == 15)
            def _():
                pltpu.sync_copy(h_hbm.at[pl.ds(p * _HALF + 4680, 320)],
                                slab_sh.at[pl.ds(4680, 320)])

            plsc.subcore_barrier()

            pltpu.sync_copy(pk_hbm.at[p, pl.ds(wid * _PKW, 8)], pk_v)
            for b in range(2):
                unpack(b, b)
                pltpu.async_copy(slab_sh.at[srcs[b]], bufs[b], sems[b])

            def pair(k, carry):
                for b in range(2):
                    j = 2 * k + b
                    pltpu.make_async_copy(
                        slab_sh.at[srcs[b]], bufs[b], sems[b]).wait()
                    pltpu.sync_copy(bufs[b], acc_sh.at[dsts[b]], add=True)
                    f = j + 2

                    @pl.when(f < _CHW)
                    def _():
                        @pl.when(f % 32 == 0)
                        def _():
                            pltpu.sync_copy(
                                pk_hbm.at[p, pl.ds(
                                    wid * _PKW + (f // 32) * 8, 8)],
                                pk_v)

                        unpack(f, b)
                        pltpu.async_copy(slab_sh.at[srcs[b]], bufs[b],
                                         sems[b])
                return carry

            lax.fori_loop(0, _CHW // 2, pair, 0)
        plsc.subcore_barrier()
        pltpu.sync_copy(acc_sh.at[pl.ds(s * _RPT, _RPT)],
                        out_hbm.at[c, pl.ds(s * _RPT, _RPT)])

    return body


_NB = 10
_BLK = _N // _NB


def _mlp_body(x_ref, p_ref, w0_ref, b0_ref, w1_ref, b1_ref, o_ref):
    h = x_ref[...] + p_ref[0] + p_ref[1]
    t = jnp.dot(h, w0_ref[...], preferred_element_type=jnp.float32)
    t = jnp.maximum(t + b0_ref[...], 0.0)
    t = jnp.dot(t, w1_ref[...], preferred_element_type=jnp.float32)
    o_ref[...] = jnp.maximum(t + b1_ref[...], 0.0)


_mlp_call = pl.pallas_call(
    _mlp_body,
    grid=(_NB,),
    in_specs=[
        pl.BlockSpec((_BLK, _D), lambda i: (i, 0)),
        pl.BlockSpec((2, _BLK, _D), lambda i: (0, i, 0)),
        pl.BlockSpec((_D, _D), lambda i: (0, 0)),
        pl.BlockSpec((1, _D), lambda i: (0, 0)),
        pl.BlockSpec((_D, _D), lambda i: (0, 0)),
        pl.BlockSpec((1, _D), lambda i: (0, 0)),
    ],
    out_specs=pl.BlockSpec((_BLK, _D), lambda i: (i, 0)),
    out_shape=jax.ShapeDtypeStruct((_N, _D), jnp.float32),
)


def _pool_body(b_ref, h_ref, w_ref, bias_ref, o_ref, acc_ref):
    i = pl.program_id(0)

    @pl.when(i == 0)
    def _():
        acc_ref[...] = jnp.zeros_like(acc_ref)

    seg = b_ref[0, 0, :]
    oh = (seg[None, :] == lax.broadcasted_iota(jnp.int32, (_G, _BLK), 0))
    acc_ref[...] += jnp.dot(oh.astype(jnp.float32), h_ref[...],
                            preferred_element_type=jnp.float32)

    @pl.when(i == _NB - 1)
    def _():
        o_ref[...] = (jnp.dot(acc_ref[...], w_ref[...],
                              preferred_element_type=jnp.float32)
                      + bias_ref[...])


_pool_call = pl.pallas_call(
    _pool_body,
    grid=(_NB,),
    in_specs=[
        pl.BlockSpec((1, 1, _BLK), lambda i: (i, 0, 0)),
        pl.BlockSpec((_BLK, _D), lambda i: (i, 0)),
        pl.BlockSpec((_D, _C), lambda i: (0, 0)),
        pl.BlockSpec((1, _C), lambda i: (0, 0)),
    ],
    out_specs=pl.BlockSpec((_G, _C), lambda i: (0, 0)),
    out_shape=jax.ShapeDtypeStruct((_G, _C), jnp.float32),
    scratch_shapes=[pltpu.VMEM((_G, _D), jnp.float32)],
)


def kernel(x, edge_index, batch, Ws, bs, gammas, betas, lin_W, lin_b):
    inv = 1.0 / jnp.sqrt(1.0 + 1e-5)
    scale = gammas * inv
    Wf = Ws * scale[:, :, None, :]
    bf = bs * scale + betas

    src = edge_index[0]
    dst = edge_index[1]
    pad = _EPAD - _E
    src_p = jnp.concatenate([src, jnp.zeros((pad,), jnp.int32)])
    dst_p = jnp.concatenate([dst, jnp.full((pad,), _TRASH, jnp.int32)])
    pks = []
    for p in range(2):
        in_half = (src_p >= p * _HALF) & (src_p < (p + 1) * _HALF)
        sloc = jnp.where(in_half, src_p - p * _HALF, 0)
        dm = jnp.where(in_half, dst_p, _TRASH)
        pks.append(sloc | (dm << 14))
    pk = jnp.stack(pks).reshape(2, _NPK, 128)
    zeros = jnp.zeros((_NACC, _D), jnp.float32)

    h = x
    for i in range(4):
        parts = _sc_edge_aggregate()(zeros, pk, h)
        h = _mlp_call(h, parts, Wf[i, 0], bf[i, 0][None],
                      Wf[i, 1], bf[i, 1][None])
    batch3 = batch.reshape(_NB, 1, _BLK)
    return _pool_call(batch3, h, lin_W, lin_b[None])
